# Initial kernel scaffold; baseline (speedup 1.0000x reference)
import functools

import jax
import jax.numpy as jnp
from jax import lax
from jax.experimental import pallas as pl
from jax.experimental.pallas import tpu as pltpu
from jax.experimental.pallas import tpu_sc as plsc

NN = 10000
NP = 10240
EE = 320000
NC = 2
NS = 16
CHUNK = 128
CPW = 79
EP = NC * NS * CPW * CHUNK
RPT = NP // NS
BLK = 1024


def _sc_mesh():
    return plsc.VectorSubcoreMesh(core_axis_name="c", subcore_axis_name="s")


def _deg_body(src_hbm, dst_hbm, outs_hbm, outd_hbm,
              idx_s, idx_d, ones_v, zero_v, accs_sh, accd_sh):
    c = lax.axis_index("c")
    s = lax.axis_index("s")
    one16 = jnp.ones((16,), jnp.float32)
    zero16 = jnp.zeros((16,), jnp.float32)

    @pl.loop(0, CHUNK)
    def _fill(r):
        ones_v[r, :] = one16
        zero_v[r, :] = zero16

    @pl.loop(0, RPT // CHUNK)
    def _zero(k):
        r0 = s * RPT + k * CHUNK
        pltpu.sync_copy(zero_v, accs_sh.at[pl.ds(r0, CHUNK)])
        pltpu.sync_copy(zero_v, accd_sh.at[pl.ds(r0, CHUNK)])

    plsc.subcore_barrier()

    row_base = (c * NS + s) * CPW
    pltpu.sync_copy(src_hbm.at[pl.ds(row_base, CPW)], idx_s)
    pltpu.sync_copy(dst_hbm.at[pl.ds(row_base, CPW)], idx_d)

    @pl.loop(0, CPW)
    def _acc(j):
        pltpu.sync_copy(ones_v, accs_sh.at[idx_s.at[j]], add=True)
        pltpu.sync_copy(ones_v, accd_sh.at[idx_d.at[j]], add=True)

    plsc.subcore_barrier()

    @pl.loop(0, RPT // CHUNK)
    def _wb(k):
        r0 = s * RPT + k * CHUNK
        pltpu.sync_copy(accs_sh.at[pl.ds(r0, CHUNK)], ones_v)
        pltpu.sync_copy(ones_v, outs_hbm.at[c, pl.ds(r0, CHUNK)])
        pltpu.sync_copy(accd_sh.at[pl.ds(r0, CHUNK)], ones_v)
        pltpu.sync_copy(ones_v, outd_hbm.at[c, pl.ds(r0, CHUNK)])


@jax.jit
def _deg_call(srcp, dstp):
    k = pl.kernel(
        _deg_body,
        out_type=(
            jax.ShapeDtypeStruct((NC, NP, 16), jnp.float32),
            jax.ShapeDtypeStruct((NC, NP, 16), jnp.float32),
        ),
        mesh=_sc_mesh(),
        scratch_types=[
            pltpu.VMEM((CPW, CHUNK), jnp.int32),
            pltpu.VMEM((CPW, CHUNK), jnp.int32),
            pltpu.VMEM((CHUNK, 16), jnp.float32),
            pltpu.VMEM((CHUNK, 16), jnp.float32),
            pltpu.VMEM_SHARED((NP, 16), jnp.float32),
            pltpu.VMEM_SHARED((NP, 16), jnp.float32),
        ],
    )
    return k(srcp, dstp)


def _spmm_body(src_hbm, dst_hbm, tbl_hbm, out_hbm,
               idx_s, idx_d, rows, tmp, tbl_sh, acc_sh):
    c = lax.axis_index("c")
    s = lax.axis_index("s")

    @pl.loop(0, RPT // CHUNK)
    def _stage(k):
        r0 = s * RPT + k * CHUNK
        pltpu.sync_copy(tbl_hbm.at[pl.ds(r0, CHUNK)], tmp)
        pltpu.sync_copy(tmp, tbl_sh.at[pl.ds(r0, CHUNK)])

    zero16 = jnp.zeros((16,), jnp.float32)

    @pl.loop(0, CHUNK)
    def _zrow(r):
        @pl.loop(0, 4)
        def _zcol(q):
            tmp[r, pl.ds(q * 16, 16)] = zero16

    @pl.loop(0, RPT // CHUNK)
    def _zero(k):
        pltpu.sync_copy(tmp, acc_sh.at[pl.ds(s * RPT + k * CHUNK, CHUNK)])

    plsc.subcore_barrier()

    row_base = (c * NS + s) * CPW
    pltpu.sync_copy(src_hbm.at[pl.ds(row_base, CPW)], idx_s)
    pltpu.sync_copy(dst_hbm.at[pl.ds(row_base, CPW)], idx_d)

    @pl.loop(0, CPW)
    def _edge(j):
        pltpu.sync_copy(tbl_sh.at[idx_s.at[j]], rows)
        pltpu.sync_copy(rows, acc_sh.at[idx_d.at[j]], add=True)

    plsc.subcore_barrier()

    @pl.loop(0, RPT // CHUNK)
    def _wb(k):
        r0 = s * RPT + k * CHUNK
        pltpu.sync_copy(acc_sh.at[pl.ds(r0, CHUNK)], tmp)
        pltpu.sync_copy(tmp, out_hbm.at[c, pl.ds(r0, CHUNK)])


@jax.jit
def _spmm_call(srcp, dstp, table):
    k = pl.kernel(
        _spmm_body,
        out_type=jax.ShapeDtypeStruct((NC, NP, 64), jnp.float32),
        mesh=_sc_mesh(),
        scratch_types=[
            pltpu.VMEM((CPW, CHUNK), jnp.int32),
            pltpu.VMEM((CPW, CHUNK), jnp.int32),
            pltpu.VMEM((CHUNK, 64), jnp.float32),
            pltpu.VMEM((CHUNK, 64), jnp.float32),
            pltpu.VMEM_SHARED((NP, 64), jnp.float32),
            pltpu.VMEM_SHARED((NP, 64), jnp.float32),
        ],
    )
    return k(srcp, dstp, table)


def _k1_body(dsrc_ref, feat_ref, w1_ref, x1_ref):
    d = dsrc_ref[0] + dsrc_ref[1]
    ns = lax.rsqrt(jnp.maximum(d[:, 0:1], 1.0))
    x1_ref[...] = jnp.dot(feat_ref[...] * ns, w1_ref[...],
                          preferred_element_type=jnp.float32)


@jax.jit
def _k1_call(dsrc, featp, W1):
    grid = (NP // BLK,)
    return pl.pallas_call(
        _k1_body,
        grid=grid,
        in_specs=[
            pl.BlockSpec((NC, BLK, 16), lambda i: (0, i, 0)),
            pl.BlockSpec((BLK, 128), lambda i: (i, 0)),
            pl.BlockSpec((128, 64), lambda i: (0, 0)),
        ],
        out_specs=pl.BlockSpec((BLK, 64), lambda i: (i, 0)),
        out_shape=jax.ShapeDtypeStruct((NP, 64), jnp.float32),
    )(dsrc, featp, W1)


def _layer_body(aggp_ref, dsrc_ref, ddst_ref, b_ref, w_ref, h_ref, xn_ref):
    agg = aggp_ref[0] + aggp_ref[1]
    dd = ddst_ref[0] + ddst_ref[1]
    nd = lax.rsqrt(jnp.maximum(dd[:, 0:1], 1.0))
    h = jnp.maximum(agg * nd + b_ref[...], 0.0)
    ds_ = dsrc_ref[0] + dsrc_ref[1]
    ns = lax.rsqrt(jnp.maximum(ds_[:, 0:1], 1.0))
    h_ref[...] = h
    xn_ref[...] = jnp.dot(h * ns, w_ref[...], preferred_element_type=jnp.float32)


@jax.jit
def _layer_call(aggp, dsrc, ddst, b, W):
    grid = (NP // BLK,)
    return pl.pallas_call(
        _layer_body,
        grid=grid,
        in_specs=[
            pl.BlockSpec((NC, BLK, 64), lambda i: (0, i, 0)),
            pl.BlockSpec((NC, BLK, 16), lambda i: (0, i, 0)),
            pl.BlockSpec((NC, BLK, 16), lambda i: (0, i, 0)),
            pl.BlockSpec((1, 64), lambda i: (0, 0)),
            pl.BlockSpec((64, 64), lambda i: (0, 0)),
        ],
        out_specs=[
            pl.BlockSpec((BLK, 64), lambda i: (i, 0)),
            pl.BlockSpec((BLK, 64), lambda i: (i, 0)),
        ],
        out_shape=[
            jax.ShapeDtypeStruct((NP, 64), jnp.float32),
            jax.ShapeDtypeStruct((NP, 64), jnp.float32),
        ],
    )(aggp, dsrc, ddst, b, W)


def _k4_body(aggp_ref, ddst_ref, b3_ref, h1_ref, h2_ref, wm_ref,
             z1_ref, z2_ref):
    agg = aggp_ref[0] + aggp_ref[1]
    dd = ddst_ref[0] + ddst_ref[1]
    nd = lax.rsqrt(jnp.maximum(dd[:, 0:1], 1.0))
    h3 = jnp.maximum(agg * nd + b3_ref[...], 0.0)
    z = (jnp.dot(h1_ref[...], wm_ref[0:64, :], preferred_element_type=jnp.float32)
         + jnp.dot(h2_ref[...], wm_ref[64:128, :], preferred_element_type=jnp.float32)
         + jnp.dot(h3, wm_ref[128:192, :], preferred_element_type=jnp.float32))
    z1_ref[...] = z[:, 0:64]
    z2_ref[...] = z[:, 64:128]


@jax.jit
def _k4_call(aggp, ddst, b3, h1, h2, Wm):
    grid = (NP // BLK,)
    return pl.pallas_call(
        _k4_body,
        grid=grid,
        in_specs=[
            pl.BlockSpec((NC, BLK, 64), lambda i: (0, i, 0)),
            pl.BlockSpec((NC, BLK, 16), lambda i: (0, i, 0)),
            pl.BlockSpec((1, 64), lambda i: (0, 0)),
            pl.BlockSpec((BLK, 64), lambda i: (i, 0)),
            pl.BlockSpec((BLK, 64), lambda i: (i, 0)),
            pl.BlockSpec((192, 128), lambda i: (0, 0)),
        ],
        out_specs=[
            pl.BlockSpec((BLK, 64), lambda i: (i, 0)),
            pl.BlockSpec((BLK, 64), lambda i: (i, 0)),
        ],
        out_shape=[
            jax.ShapeDtypeStruct((NP, 64), jnp.float32),
            jax.ShapeDtypeStruct((NP, 64), jnp.float32),
        ],
    )(aggp, ddst, b3, h1, h2, Wm)


def _k5_body(p1_ref, p2_ref, bm_ref, out_ref):
    a = p1_ref[0] + p1_ref[1]
    b = p2_ref[0] + p2_ref[1]
    out_ref[...] = jnp.concatenate([a, b], axis=1) + bm_ref[...]


@jax.jit
def _k5_call(p1, p2, bm):
    grid = (NP // BLK,)
    return pl.pallas_call(
        _k5_body,
        grid=grid,
        in_specs=[
            pl.BlockSpec((NC, BLK, 64), lambda i: (0, i, 0)),
            pl.BlockSpec((NC, BLK, 64), lambda i: (0, i, 0)),
            pl.BlockSpec((1, 128), lambda i: (0, 0)),
        ],
        out_specs=pl.BlockSpec((BLK, 128), lambda i: (i, 0)),
        out_shape=jax.ShapeDtypeStruct((NP, 128), jnp.float32),
    )(p1, p2, bm)


@jax.jit
def kernel(feat, edge_index, W1, b1, W2, b2, W3, b3, Wm, bm):
    src = edge_index[0]
    dst = edge_index[1]
    pad_n = EP - EE
    pad_idx = NN + (jnp.arange(pad_n, dtype=jnp.int32) % (NP - NN))
    srcp = jnp.concatenate([src, pad_idx]).reshape(EP // CHUNK, CHUNK)
    dstp = jnp.concatenate([dst, pad_idx]).reshape(EP // CHUNK, CHUNK)
    featp = jnp.pad(feat, ((0, NP - NN), (0, 0)))

    dsrc, ddst = _deg_call(srcp, dstp)
    x1 = _k1_call(dsrc, featp, W1)
    a1 = _spmm_call(srcp, dstp, x1)
    h1, x2 = _layer_call(a1, dsrc, ddst, b1.reshape(1, 64), W2)
    a2 = _spmm_call(srcp, dstp, x2)
    h2, x3 = _layer_call(a2, dsrc, ddst, b2.reshape(1, 64), W3)
    a3 = _spmm_call(srcp, dstp, x3)
    z1, z2 = _k4_call(a3, ddst, b3.reshape(1, 64), h1, h2, Wm)
    p1 = _spmm_call(srcp, dstp, z1)
    p2 = _spmm_call(srcp, dstp, z2)
    out = _k5_call(p1, p2, bm.reshape(1, 128))
    return out[:NN]


# speedup vs baseline: 4.7929x; 4.7929x over previous
import dataclasses

import jax
import jax.numpy as jnp
from jax import lax
from jax.experimental import pallas as pl
from jax.experimental.pallas import tpu as pltpu
from jax.experimental.pallas import tpu_sc as plsc

NN = 10000
NP = 10240
EE = 320000
NC = 2
NS = 16
CHUNK = 128
CPW = 80
EP = NC * NS * CPW * CHUNK
RPT = NP // NS
HR = NP // 16
BLK = 1024


def _sc_mesh():
    return plsc.VectorSubcoreMesh(core_axis_name="c", subcore_axis_name="s")


def _sc_params():
    cp = pltpu.CompilerParams()
    if "needs_layout_passes" in pltpu.CompilerParams.__dataclass_fields__:
        cp = dataclasses.replace(cp, needs_layout_passes=False)
    return cp


def _deg_body(src_hbm, dst_hbm, zin_hbm, io_hbm, outs_hbm, outd_hbm,
              idx_v, iid, hist, accs_sh, accd_sh):
    c = lax.axis_index("c")
    s = lax.axis_index("s")
    lane = jnp.arange(16, dtype=jnp.int32)
    ones16 = jnp.ones((16,), jnp.float32)

    pltpu.sync_copy(zin_hbm, hist.at[pl.ds(0, CHUNK)])
    pltpu.sync_copy(hist.at[pl.ds(0, 80)], accs_sh.at[pl.ds(s * 80, 80)])
    pltpu.sync_copy(hist.at[pl.ds(0, 80)], accd_sh.at[pl.ds(s * 80, 80)])
    plsc.subcore_barrier()

    edge_base = (c * NS + s) * CPW * CHUNK

    def one_hist(sel_hbm, acc_sel):
        for r in range(2):
            @pl.loop(0, HR // CHUNK)
            def _hz(k):
                pltpu.sync_copy(zin_hbm, hist.at[pl.ds(k * CHUNK, CHUNK)])

            @pl.loop(0, CPW)
            def _chunk(j):
                pltpu.sync_copy(sel_hbm.at[pl.ds(edge_base + j * CHUNK, CHUNK)],
                                idx_v)
                for q in range(8):
                    vec = idx_v[pl.ds(q * 16, 16)]
                    m = vec - (r * 5120)
                    mask = (m >= 0) & (m < 5120)
                    mm = jnp.where(mask, m, 0)
                    vrow = lax.shift_right_logical(mm, 3)
                    vcol = lax.shift_left(lax.bitwise_and(mm, 7), 4) + lane
                    plsc.addupdate_scatter(hist, [vrow, vcol], ones16,
                                           mask=mask)

            @pl.loop(0, HR // CHUNK)
            def _flush(k):
                pltpu.sync_copy(io_hbm.at[pl.ds(r * HR + k * CHUNK, CHUNK)],
                                iid)
                pltpu.sync_copy(hist.at[pl.ds(k * CHUNK, CHUNK)],
                                acc_sel.at[iid], add=True)

    one_hist(src_hbm, accs_sh)
    one_hist(dst_hbm, accd_sh)
    plsc.subcore_barrier()

    pltpu.sync_copy(accs_sh.at[pl.ds(s * 80, 80)], hist.at[pl.ds(0, 80)])
    pltpu.sync_copy(hist.at[pl.ds(0, 80)],
                    outs_hbm.at[pl.ds(c * (2 * HR) + s * 80, 80)])
    pltpu.sync_copy(accd_sh.at[pl.ds(s * 80, 80)], hist.at[pl.ds(0, 80)])
    pltpu.sync_copy(hist.at[pl.ds(0, 80)],
                    outd_hbm.at[pl.ds(c * (2 * HR) + s * 80, 80)])


@jax.jit
def _deg_call(srcf, dstf, zin, io):
    k = pl.kernel(
        _deg_body,
        out_type=(
            jax.ShapeDtypeStruct((NC * 2 * HR, CHUNK), jnp.float32),
            jax.ShapeDtypeStruct((NC * 2 * HR, CHUNK), jnp.float32),
        ),
        mesh=_sc_mesh(),
        scratch_types=[
            pltpu.VMEM((CHUNK,), jnp.int32),
            pltpu.VMEM((CHUNK,), jnp.int32),
            pltpu.VMEM((HR, CHUNK), jnp.float32),
            pltpu.VMEM_SHARED((2 * HR, CHUNK), jnp.float32),
            pltpu.VMEM_SHARED((2 * HR, CHUNK), jnp.float32),
        ],
        compiler_params=_sc_params(),
    )
    o1, o2 = k(srcf, dstf, zin, io)
    return (o1.reshape(NC, 2 * HR, CHUNK), o2.reshape(NC, 2 * HR, CHUNK))


def _spmm_body(src_hbm, dst_hbm, tbl_hbm, zin_hbm, out_hbm,
               idx_s, idx_d, rows, acc_sh):
    c = lax.axis_index("c")
    s = lax.axis_index("s")

    pltpu.sync_copy(zin_hbm, rows)

    @pl.loop(0, RPT // CHUNK)
    def _zero(k):
        pltpu.sync_copy(rows, acc_sh.at[pl.ds(s * RPT + k * CHUNK, CHUNK)])

    plsc.subcore_barrier()

    edge_base = (c * NS + s) * CPW * CHUNK

    @pl.loop(0, CPW)
    def _edge(j):
        e0 = edge_base + j * CHUNK
        pltpu.sync_copy(src_hbm.at[pl.ds(e0, CHUNK)], idx_s)
        pltpu.sync_copy(dst_hbm.at[pl.ds(e0, CHUNK)], idx_d)
        pltpu.sync_copy(tbl_hbm.at[idx_s], rows)
        pltpu.sync_copy(rows, acc_sh.at[idx_d], add=True)

    plsc.subcore_barrier()

    @pl.loop(0, RPT // CHUNK)
    def _wb(k):
        r0 = s * RPT + k * CHUNK
        pltpu.sync_copy(acc_sh.at[pl.ds(r0, CHUNK)], rows)
        pltpu.sync_copy(rows, out_hbm.at[pl.ds(c * NP + r0, CHUNK)])


@jax.jit
def _spmm_call(srcf, dstf, table, zin):
    k = pl.kernel(
        _spmm_body,
        out_type=jax.ShapeDtypeStruct((NC * NP, CHUNK), jnp.float32),
        mesh=_sc_mesh(),
        scratch_types=[
            pltpu.VMEM((CHUNK,), jnp.int32),
            pltpu.VMEM((CHUNK,), jnp.int32),
            pltpu.VMEM((CHUNK, CHUNK), jnp.float32),
            pltpu.VMEM_SHARED((NP, CHUNK), jnp.float32),
        ],
    )
    return k(srcf, dstf, table, zin).reshape(NC, NP, CHUNK)


def _deg_block(dp_ref):
    d = dp_ref[0] + dp_ref[1]
    sel = (lax.broadcasted_iota(jnp.int32, (128, 8), 0) // 16
           == lax.broadcasted_iota(jnp.int32, (128, 8), 1)
           ).astype(jnp.float32)
    return jnp.dot(d, sel, preferred_element_type=jnp.float32)


def _scale_rows(x, n38):
    w = x.shape[1]
    return (x.reshape(128, 8, w) * n38[:, :, None]).reshape(1024, w)


def _norm38(dp_ref):
    return lax.rsqrt(jnp.maximum(_deg_block(dp_ref), 1.0))


def _k1_body(dsp_ref, feat_ref, t1_ref):
    ns = _norm38(dsp_ref)
    t1_ref[...] = _scale_rows(feat_ref[...], ns)


@jax.jit
def _k1_call(dsp, featp):
    return pl.pallas_call(
        _k1_body,
        grid=(NP // BLK,),
        in_specs=[
            pl.BlockSpec((NC, 128, 128), lambda i: (0, i, 0)),
            pl.BlockSpec((BLK, 128), lambda i: (i, 0)),
        ],
        out_specs=pl.BlockSpec((BLK, 128), lambda i: (i, 0)),
        out_shape=jax.ShapeDtypeStruct((NP, 128), jnp.float32),
    )(dsp, featp)


def _k2_body(p_ref, dsp_ref, ddp_ref, w1_ref, b1_ref, w2_ref, t2_ref):
    nd = _norm38(ddp_ref)
    ns = _norm38(dsp_ref)
    aggf = _scale_rows(p_ref[0] + p_ref[1], nd)
    h1 = jnp.maximum(
        jnp.dot(aggf, w1_ref[...], preferred_element_type=jnp.float32)
        + b1_ref[...], 0.0)
    x2 = jnp.dot(_scale_rows(h1, ns), w2_ref[...],
                 preferred_element_type=jnp.float32)
    t2_ref[...] = jnp.concatenate([x2, h1], axis=1)


@jax.jit
def _k2_call(p1, dsp, ddp, W1, b1, W2):
    return pl.pallas_call(
        _k2_body,
        grid=(NP // BLK,),
        in_specs=[
            pl.BlockSpec((NC, BLK, 128), lambda i: (0, i, 0)),
            pl.BlockSpec((NC, 128, 128), lambda i: (0, i, 0)),
            pl.BlockSpec((NC, 128, 128), lambda i: (0, i, 0)),
            pl.BlockSpec((128, 64), lambda i: (0, 0)),
            pl.BlockSpec((1, 64), lambda i: (0, 0)),
            pl.BlockSpec((64, 64), lambda i: (0, 0)),
        ],
        out_specs=pl.BlockSpec((BLK, 128), lambda i: (i, 0)),
        out_shape=jax.ShapeDtypeStruct((NP, 128), jnp.float32),
    )(p1, dsp, ddp, W1, b1, W2)


def _k3_body(p_ref, dsp_ref, ddp_ref, b2_ref, w3_ref, t3_ref):
    nd = _norm38(ddp_ref)
    ns = _norm38(dsp_ref)
    a = p_ref[0] + p_ref[1]
    h2 = jnp.maximum(_scale_rows(a[:, 0:64], nd) + b2_ref[...], 0.0)
    x3 = jnp.dot(_scale_rows(h2, ns), w3_ref[...],
                 preferred_element_type=jnp.float32)
    t3_ref[...] = jnp.concatenate([x3, h2], axis=1)


@jax.jit
def _k3_call(p2, dsp, ddp, b2, W3):
    return pl.pallas_call(
        _k3_body,
        grid=(NP // BLK,),
        in_specs=[
            pl.BlockSpec((NC, BLK, 128), lambda i: (0, i, 0)),
            pl.BlockSpec((NC, 128, 128), lambda i: (0, i, 0)),
            pl.BlockSpec((NC, 128, 128), lambda i: (0, i, 0)),
            pl.BlockSpec((1, 64), lambda i: (0, 0)),
            pl.BlockSpec((64, 64), lambda i: (0, 0)),
        ],
        out_specs=pl.BlockSpec((BLK, 128), lambda i: (i, 0)),
        out_shape=jax.ShapeDtypeStruct((NP, 128), jnp.float32),
    )(p2, dsp, ddp, b2, W3)


def _k4_body(p_ref, ddp_ref, b3_ref, wm_ref, t4_ref):
    nd = _norm38(ddp_ref)
    a = p_ref[0] + p_ref[1]
    h3 = jnp.maximum(_scale_rows(a[:, 0:64], nd) + b3_ref[...], 0.0)
    t4_ref[...] = jnp.dot(h3, wm_ref[128:192, :],
                          preferred_element_type=jnp.float32)


@jax.jit
def _k4_call(p3, ddp, b3, Wm):
    return pl.pallas_call(
        _k4_body,
        grid=(NP // BLK,),
        in_specs=[
            pl.BlockSpec((NC, BLK, 128), lambda i: (0, i, 0)),
            pl.BlockSpec((NC, 128, 128), lambda i: (0, i, 0)),
            pl.BlockSpec((1, 64), lambda i: (0, 0)),
            pl.BlockSpec((192, 128), lambda i: (0, 0)),
        ],
        out_specs=pl.BlockSpec((BLK, 128), lambda i: (i, 0)),
        out_shape=jax.ShapeDtypeStruct((NP, 128), jnp.float32),
    )(p3, ddp, b3, Wm)


def _k5_body(p2_ref, p3_ref, p4_ref, wm_ref, bm_ref, out_ref):
    j1 = p2_ref[0, :, 64:128] + p2_ref[1, :, 64:128]
    j2 = p3_ref[0, :, 64:128] + p3_ref[1, :, 64:128]
    sz3 = p4_ref[0] + p4_ref[1]
    out_ref[...] = (
        jnp.dot(j1, wm_ref[0:64, :], preferred_element_type=jnp.float32)
        + jnp.dot(j2, wm_ref[64:128, :], preferred_element_type=jnp.float32)
        + sz3 + bm_ref[...])


@jax.jit
def _k5_call(p2, p3, p4, Wm, bm):
    return pl.pallas_call(
        _k5_body,
        grid=(NP // BLK,),
        in_specs=[
            pl.BlockSpec((NC, BLK, 128), lambda i: (0, i, 0)),
            pl.BlockSpec((NC, BLK, 128), lambda i: (0, i, 0)),
            pl.BlockSpec((NC, BLK, 128), lambda i: (0, i, 0)),
            pl.BlockSpec((192, 128), lambda i: (0, 0)),
            pl.BlockSpec((1, 128), lambda i: (0, 0)),
        ],
        out_specs=pl.BlockSpec((BLK, 128), lambda i: (i, 0)),
        out_shape=jax.ShapeDtypeStruct((NP, 128), jnp.float32),
    )(p2, p3, p4, Wm, bm)


@jax.jit
def kernel(feat, edge_index, W1, b1, W2, b2, W3, b3, Wm, bm):
    src = edge_index[0]
    dst = edge_index[1]
    pad_n = EP - EE
    pad_idx = NN + (jnp.arange(pad_n, dtype=jnp.int32) % (NP - NN))
    srcf = jnp.concatenate([src, pad_idx])
    dstf = jnp.concatenate([dst, pad_idx])
    featp = jnp.pad(feat, ((0, NP - NN), (0, 0)))
    zin = jnp.zeros((CHUNK, CHUNK), jnp.float32)
    io = jnp.arange(2 * HR, dtype=jnp.int32)

    dsp, ddp = _deg_call(srcf, dstf, zin, io)
    t1 = _k1_call(dsp, featp)
    p1 = _spmm_call(srcf, dstf, t1, zin)
    t2 = _k2_call(p1, dsp, ddp, W1, b1.reshape(1, 64), W2)
    p2 = _spmm_call(srcf, dstf, t2, zin)
    t3 = _k3_call(p2, dsp, ddp, b2.reshape(1, 64), W3)
    p3 = _spmm_call(srcf, dstf, t3, zin)
    t4 = _k4_call(p3, ddp, b3.reshape(1, 64), Wm)
    p4 = _spmm_call(srcf, dstf, t4, zin)
    out = _k5_call(p2, p3, p4, Wm, bm.reshape(1, 128))
    return out[:NN]


# speedup vs baseline: 7.5754x; 1.5805x over previous
import dataclasses

import jax
import jax.numpy as jnp
from jax import lax
from jax.experimental import pallas as pl
from jax.experimental.pallas import tpu as pltpu
from jax.experimental.pallas import tpu_sc as plsc

NN = 10000
NP = 10240
EE = 320000
NC = 2
NS = 16
CHUNK = 128
CPW = 80
EP = NC * NS * CPW * CHUNK
RPT = NP // NS
HR = NP // 16
BLK = 1024


def _sc_mesh():
    return plsc.VectorSubcoreMesh(core_axis_name="c", subcore_axis_name="s")


def _sc_params():
    cp = pltpu.CompilerParams()
    if "needs_layout_passes" in pltpu.CompilerParams.__dataclass_fields__:
        cp = dataclasses.replace(cp, needs_layout_passes=False)
    return cp


def _deg_body(src_hbm, dst_hbm, zin_hbm, io_hbm, outs_hbm, outd_hbm,
              idx8, iid, hist, accs_sh, accd_sh):
    c = lax.axis_index("c")
    s = lax.axis_index("s")
    lane = jnp.arange(16, dtype=jnp.int32)
    ones16 = jnp.ones((16,), jnp.float32)

    pltpu.sync_copy(zin_hbm, hist.at[pl.ds(0, CHUNK)])
    pltpu.sync_copy(hist.at[pl.ds(0, 80)], accs_sh.at[pl.ds(s * 80, 80)])
    pltpu.sync_copy(hist.at[pl.ds(0, 80)], accd_sh.at[pl.ds(s * 80, 80)])
    plsc.subcore_barrier()

    row_base = (c * NS + s) * CPW

    def one_hist(sel_hbm, acc_sel):
        for r in range(2):
            @pl.loop(0, HR // CHUNK)
            def _hz(k):
                pltpu.sync_copy(zin_hbm, hist.at[pl.ds(k * CHUNK, CHUNK)])

            @pl.loop(0, CPW // 8)
            def _blk(b):
                pltpu.sync_copy(sel_hbm.at[pl.ds(row_base + b * 8, 8)], idx8)
                for jj in range(8):
                    for q in range(8):
                        vec = idx8[jj, pl.ds(q * 16, 16)]
                        m = vec - (r * 5120)
                        mask = (m >= 0) & (m < 5120)
                        mm = jnp.where(mask, m, 0)
                        vrow = lax.shift_right_logical(mm, 3)
                        vcol = lax.shift_left(lax.bitwise_and(mm, 7), 4) + lane
                        plsc.addupdate_scatter(hist, [vrow, vcol], ones16,
                                               mask=mask)

            @pl.loop(0, HR // CHUNK)
            def _flush(k):
                pltpu.sync_copy(io_hbm.at[pl.ds(r * HR + k * CHUNK, CHUNK)],
                                iid)
                pltpu.sync_copy(hist.at[pl.ds(k * CHUNK, CHUNK)],
                                acc_sel.at[iid], add=True)

    one_hist(src_hbm, accs_sh)
    one_hist(dst_hbm, accd_sh)
    plsc.subcore_barrier()

    pltpu.sync_copy(accs_sh.at[pl.ds(s * 80, 80)], hist.at[pl.ds(0, 80)])
    pltpu.sync_copy(hist.at[pl.ds(0, 80)],
                    outs_hbm.at[pl.ds(c * (2 * HR) + s * 80, 80)])
    pltpu.sync_copy(accd_sh.at[pl.ds(s * 80, 80)], hist.at[pl.ds(0, 80)])
    pltpu.sync_copy(hist.at[pl.ds(0, 80)],
                    outd_hbm.at[pl.ds(c * (2 * HR) + s * 80, 80)])


@jax.jit
def _deg_call(srcf, dstf, zin, io):
    k = pl.kernel(
        _deg_body,
        out_type=(
            jax.ShapeDtypeStruct((NC * 2 * HR, CHUNK), jnp.float32),
            jax.ShapeDtypeStruct((NC * 2 * HR, CHUNK), jnp.float32),
        ),
        mesh=_sc_mesh(),
        scratch_types=[
            pltpu.VMEM((8, CHUNK), jnp.int32),
            pltpu.VMEM((CHUNK,), jnp.int32),
            pltpu.VMEM((HR, CHUNK), jnp.float32),
            pltpu.VMEM_SHARED((2 * HR, CHUNK), jnp.float32),
            pltpu.VMEM_SHARED((2 * HR, CHUNK), jnp.float32),
        ],
        compiler_params=_sc_params(),
    )
    o1, o2 = k(srcf, dstf, zin, io)
    return (o1.reshape(NC, 2 * HR, CHUNK), o2.reshape(NC, 2 * HR, CHUNK))


def _spmm_body(src_hbm, dst_hbm, tbl_hbm, zin_hbm, out_hbm,
               idx_s8, idx_d8, rows_a, rows_b, acc_sh,
               isem_s, isem_d, gsem_a, gsem_b, ssem_a, ssem_b):
    c = lax.axis_index("c")
    s = lax.axis_index("s")

    pltpu.sync_copy(zin_hbm, rows_a)

    @pl.loop(0, RPT // CHUNK)
    def _zero(k):
        pltpu.sync_copy(rows_a, acc_sh.at[pl.ds(s * RPT + k * CHUNK, CHUNK)])

    plsc.subcore_barrier()

    row_base = (c * NS + s) * CPW

    @pl.loop(0, CPW // 8)
    def _blk(b):
        r0 = row_base + b * 8
        i1 = pltpu.async_copy(src_hbm.at[pl.ds(r0, 8)], idx_s8, isem_s)
        i2 = pltpu.async_copy(dst_hbm.at[pl.ds(r0, 8)], idx_d8, isem_d)
        i1.wait()
        i2.wait()
        sa = sb = None
        for u in range(4):
            if sa is not None:
                sa.wait()
                sb.wait()
            ga = pltpu.async_copy(tbl_hbm.at[idx_s8.at[2 * u]], rows_a, gsem_a)
            gb = pltpu.async_copy(tbl_hbm.at[idx_s8.at[2 * u + 1]], rows_b,
                                  gsem_b)
            ga.wait()
            sa = pltpu.async_copy(rows_a, acc_sh.at[idx_d8.at[2 * u]], ssem_a,
                                  add=True)
            gb.wait()
            sb = pltpu.async_copy(rows_b, acc_sh.at[idx_d8.at[2 * u + 1]],
                                  ssem_b, add=True)
        sa.wait()
        sb.wait()

    plsc.subcore_barrier()

    @pl.loop(0, RPT // CHUNK)
    def _wb(k):
        r0 = s * RPT + k * CHUNK
        pltpu.sync_copy(acc_sh.at[pl.ds(r0, CHUNK)], rows_a)
        pltpu.sync_copy(rows_a, out_hbm.at[pl.ds(c * NP + r0, CHUNK)])


@jax.jit
def _spmm_call(src2d, dst2d, table, zin):
    k = pl.kernel(
        _spmm_body,
        out_type=jax.ShapeDtypeStruct((NC * NP, CHUNK), jnp.float32),
        mesh=_sc_mesh(),
        scratch_types=[
            pltpu.VMEM((8, CHUNK), jnp.int32),
            pltpu.VMEM((8, CHUNK), jnp.int32),
            pltpu.VMEM((CHUNK, CHUNK), jnp.float32),
            pltpu.VMEM((CHUNK, CHUNK), jnp.float32),
            pltpu.VMEM_SHARED((NP, CHUNK), jnp.float32),
            pltpu.SemaphoreType.DMA,
            pltpu.SemaphoreType.DMA,
            pltpu.SemaphoreType.DMA,
            pltpu.SemaphoreType.DMA,
            pltpu.SemaphoreType.DMA,
            pltpu.SemaphoreType.DMA,
        ],
    )
    return k(src2d, dst2d, table, zin).reshape(NC, NP, CHUNK)


def _deg_block(dp_ref):
    d = dp_ref[0] + dp_ref[1]
    sel = (lax.broadcasted_iota(jnp.int32, (128, 8), 0) // 16
           == lax.broadcasted_iota(jnp.int32, (128, 8), 1)
           ).astype(jnp.float32)
    return jnp.dot(d, sel, preferred_element_type=jnp.float32)


def _scale_rows(x, n38):
    w = x.shape[1]
    return (x.reshape(128, 8, w) * n38[:, :, None]).reshape(1024, w)


def _norm38(dp_ref):
    return lax.rsqrt(jnp.maximum(_deg_block(dp_ref), 1.0))


def _k1_body(dsp_ref, feat_ref, t1_ref):
    ns = _norm38(dsp_ref)
    t1_ref[...] = _scale_rows(feat_ref[...], ns)


@jax.jit
def _k1_call(dsp, featp):
    return pl.pallas_call(
        _k1_body,
        grid=(NP // BLK,),
        in_specs=[
            pl.BlockSpec((NC, 128, 128), lambda i: (0, i, 0)),
            pl.BlockSpec((BLK, 128), lambda i: (i, 0)),
        ],
        out_specs=pl.BlockSpec((BLK, 128), lambda i: (i, 0)),
        out_shape=jax.ShapeDtypeStruct((NP, 128), jnp.float32),
    )(dsp, featp)


def _k2_body(p_ref, dsp_ref, ddp_ref, w1_ref, b1_ref, w2_ref, t2_ref):
    nd = _norm38(ddp_ref)
    ns = _norm38(dsp_ref)
    aggf = _scale_rows(p_ref[0] + p_ref[1], nd)
    h1 = jnp.maximum(
        jnp.dot(aggf, w1_ref[...], preferred_element_type=jnp.float32)
        + b1_ref[...], 0.0)
    x2 = jnp.dot(_scale_rows(h1, ns), w2_ref[...],
                 preferred_element_type=jnp.float32)
    t2_ref[...] = jnp.concatenate([x2, h1], axis=1)


@jax.jit
def _k2_call(p1, dsp, ddp, W1, b1, W2):
    return pl.pallas_call(
        _k2_body,
        grid=(NP // BLK,),
        in_specs=[
            pl.BlockSpec((NC, BLK, 128), lambda i: (0, i, 0)),
            pl.BlockSpec((NC, 128, 128), lambda i: (0, i, 0)),
            pl.BlockSpec((NC, 128, 128), lambda i: (0, i, 0)),
            pl.BlockSpec((128, 64), lambda i: (0, 0)),
            pl.BlockSpec((1, 64), lambda i: (0, 0)),
            pl.BlockSpec((64, 64), lambda i: (0, 0)),
        ],
        out_specs=pl.BlockSpec((BLK, 128), lambda i: (i, 0)),
        out_shape=jax.ShapeDtypeStruct((NP, 128), jnp.float32),
    )(p1, dsp, ddp, W1, b1, W2)


def _k3_body(p_ref, dsp_ref, ddp_ref, b2_ref, w3_ref, t3_ref):
    nd = _norm38(ddp_ref)
    ns = _norm38(dsp_ref)
    a = p_ref[0] + p_ref[1]
    h2 = jnp.maximum(_scale_rows(a[:, 0:64], nd) + b2_ref[...], 0.0)
    x3 = jnp.dot(_scale_rows(h2, ns), w3_ref[...],
                 preferred_element_type=jnp.float32)
    t3_ref[...] = jnp.concatenate([x3, h2], axis=1)


@jax.jit
def _k3_call(p2, dsp, ddp, b2, W3):
    return pl.pallas_call(
        _k3_body,
        grid=(NP // BLK,),
        in_specs=[
            pl.BlockSpec((NC, BLK, 128), lambda i: (0, i, 0)),
            pl.BlockSpec((NC, 128, 128), lambda i: (0, i, 0)),
            pl.BlockSpec((NC, 128, 128), lambda i: (0, i, 0)),
            pl.BlockSpec((1, 64), lambda i: (0, 0)),
            pl.BlockSpec((64, 64), lambda i: (0, 0)),
        ],
        out_specs=pl.BlockSpec((BLK, 128), lambda i: (i, 0)),
        out_shape=jax.ShapeDtypeStruct((NP, 128), jnp.float32),
    )(p2, dsp, ddp, b2, W3)


def _k4_body(p_ref, ddp_ref, b3_ref, wm_ref, t4_ref):
    nd = _norm38(ddp_ref)
    a = p_ref[0] + p_ref[1]
    h3 = jnp.maximum(_scale_rows(a[:, 0:64], nd) + b3_ref[...], 0.0)
    t4_ref[...] = jnp.dot(h3, wm_ref[128:192, :],
                          preferred_element_type=jnp.float32)


@jax.jit
def _k4_call(p3, ddp, b3, Wm):
    return pl.pallas_call(
        _k4_body,
        grid=(NP // BLK,),
        in_specs=[
            pl.BlockSpec((NC, BLK, 128), lambda i: (0, i, 0)),
            pl.BlockSpec((NC, 128, 128), lambda i: (0, i, 0)),
            pl.BlockSpec((1, 64), lambda i: (0, 0)),
            pl.BlockSpec((192, 128), lambda i: (0, 0)),
        ],
        out_specs=pl.BlockSpec((BLK, 128), lambda i: (i, 0)),
        out_shape=jax.ShapeDtypeStruct((NP, 128), jnp.float32),
    )(p3, ddp, b3, Wm)


def _k5_body(p2_ref, p3_ref, p4_ref, wm_ref, bm_ref, out_ref):
    j1 = p2_ref[0, :, 64:128] + p2_ref[1, :, 64:128]
    j2 = p3_ref[0, :, 64:128] + p3_ref[1, :, 64:128]
    sz3 = p4_ref[0] + p4_ref[1]
    out_ref[...] = (
        jnp.dot(j1, wm_ref[0:64, :], preferred_element_type=jnp.float32)
        + jnp.dot(j2, wm_ref[64:128, :], preferred_element_type=jnp.float32)
        + sz3 + bm_ref[...])


@jax.jit
def _k5_call(p2, p3, p4, Wm, bm):
    return pl.pallas_call(
        _k5_body,
        grid=(NP // BLK,),
        in_specs=[
            pl.BlockSpec((NC, BLK, 128), lambda i: (0, i, 0)),
            pl.BlockSpec((NC, BLK, 128), lambda i: (0, i, 0)),
            pl.BlockSpec((NC, BLK, 128), lambda i: (0, i, 0)),
            pl.BlockSpec((192, 128), lambda i: (0, 0)),
            pl.BlockSpec((1, 128), lambda i: (0, 0)),
        ],
        out_specs=pl.BlockSpec((BLK, 128), lambda i: (i, 0)),
        out_shape=jax.ShapeDtypeStruct((NP, 128), jnp.float32),
    )(p2, p3, p4, Wm, bm)


@jax.jit
def kernel(feat, edge_index, W1, b1, W2, b2, W3, b3, Wm, bm):
    src = edge_index[0]
    dst = edge_index[1]
    pad_n = EP - EE
    pad_idx = NN + (jnp.arange(pad_n, dtype=jnp.int32) % (NP - NN))
    src2d = jnp.concatenate([src, pad_idx]).reshape(EP // CHUNK, CHUNK)
    dst2d = jnp.concatenate([dst, pad_idx]).reshape(EP // CHUNK, CHUNK)
    featp = jnp.pad(feat, ((0, NP - NN), (0, 0)))
    zin = jnp.zeros((CHUNK, CHUNK), jnp.float32)
    io = jnp.arange(2 * HR, dtype=jnp.int32)

    dsp, ddp = _deg_call(src2d, dst2d, zin, io)
    t1 = _k1_call(dsp, featp)
    p1 = _spmm_call(src2d, dst2d, t1, zin)
    t2 = _k2_call(p1, dsp, ddp, W1, b1.reshape(1, 64), W2)
    p2 = _spmm_call(src2d, dst2d, t2, zin)
    t3 = _k3_call(p2, dsp, ddp, b2.reshape(1, 64), W3)
    p3 = _spmm_call(src2d, dst2d, t3, zin)
    t4 = _k4_call(p3, ddp, b3.reshape(1, 64), Wm)
    p4 = _spmm_call(src2d, dst2d, t4, zin)
    out = _k5_call(p2, p3, p4, Wm, bm.reshape(1, 128))
    return out[:NN]


# speedup vs baseline: 8.5168x; 1.1243x over previous
import dataclasses

import jax
import jax.numpy as jnp
from jax import lax
from jax.experimental import pallas as pl
from jax.experimental.pallas import tpu as pltpu
from jax.experimental.pallas import tpu_sc as plsc

NN = 10000
NP = 10240
EE = 320000
NC = 2
NS = 16
CHUNK = 128
CPW = 80
EP = NC * NS * CPW * CHUNK
RPT = NP // NS
HR = NP // 16
BLK = 1024


def _sc_mesh():
    return plsc.VectorSubcoreMesh(core_axis_name="c", subcore_axis_name="s")


def _sc_params():
    cp = pltpu.CompilerParams()
    if "needs_layout_passes" in pltpu.CompilerParams.__dataclass_fields__:
        cp = dataclasses.replace(cp, needs_layout_passes=False)
    return cp


def _deg_body(src_hbm, dst_hbm, zin_hbm, io_hbm, outs_hbm, outd_hbm,
              idx_all_s, idx_all_d, iid, hist, accs_sh, accd_sh):
    c = lax.axis_index("c")
    s = lax.axis_index("s")
    lane = jnp.arange(16, dtype=jnp.int32)
    ones16 = jnp.ones((16,), jnp.float32)

    pltpu.sync_copy(zin_hbm.at[pl.ds(0, 80)], accs_sh.at[pl.ds(s * 80, 80)])
    pltpu.sync_copy(zin_hbm.at[pl.ds(0, 80)], accd_sh.at[pl.ds(s * 80, 80)])
    row_base = (c * NS + s) * CPW
    pltpu.sync_copy(src_hbm.at[pl.ds(row_base, CPW)], idx_all_s)
    pltpu.sync_copy(dst_hbm.at[pl.ds(row_base, CPW)], idx_all_d)
    plsc.subcore_barrier()

    def one_hist(idx_all, acc_sel):
        for r in range(2):
            @pl.loop(0, HR // CHUNK)
            def _hz(k):
                pltpu.sync_copy(zin_hbm, hist.at[pl.ds(k * CHUNK, CHUNK)])

            @pl.loop(0, CPW // 8)
            def _blk(b):
                for jj in range(8):
                    for q in range(8):
                        vec = idx_all[b * 8 + jj, pl.ds(q * 16, 16)]
                        m = vec - (r * 5120)
                        mask = (m >= 0) & (m < 5120)
                        mm = jnp.where(mask, m, 0)
                        vrow = lax.shift_right_logical(mm, 3)
                        vcol = lax.shift_left(lax.bitwise_and(mm, 7), 4) + lane
                        plsc.addupdate_scatter(hist, [vrow, vcol], ones16,
                                               mask=mask)

            @pl.loop(0, HR // CHUNK)
            def _flush(k):
                pltpu.sync_copy(io_hbm.at[pl.ds(r * HR + k * CHUNK, CHUNK)],
                                iid)
                pltpu.sync_copy(hist.at[pl.ds(k * CHUNK, CHUNK)],
                                acc_sel.at[iid], add=True)

    one_hist(idx_all_s, accs_sh)
    one_hist(idx_all_d, accd_sh)
    plsc.subcore_barrier()

    pltpu.sync_copy(accs_sh.at[pl.ds(s * 80, 80)],
                    outs_hbm.at[pl.ds(c * (2 * HR) + s * 80, 80)])
    pltpu.sync_copy(accd_sh.at[pl.ds(s * 80, 80)],
                    outd_hbm.at[pl.ds(c * (2 * HR) + s * 80, 80)])


@jax.jit
def _deg_call(srcf, dstf, zin, io):
    k = pl.kernel(
        _deg_body,
        out_type=(
            jax.ShapeDtypeStruct((NC * 2 * HR, CHUNK), jnp.float32),
            jax.ShapeDtypeStruct((NC * 2 * HR, CHUNK), jnp.float32),
        ),
        mesh=_sc_mesh(),
        scratch_types=[
            pltpu.VMEM((CPW, CHUNK), jnp.int32),
            pltpu.VMEM((CPW, CHUNK), jnp.int32),
            pltpu.VMEM((CHUNK,), jnp.int32),
            pltpu.VMEM((HR, CHUNK), jnp.float32),
            pltpu.VMEM_SHARED((2 * HR, CHUNK), jnp.float32),
            pltpu.VMEM_SHARED((2 * HR, CHUNK), jnp.float32),
        ],
        compiler_params=_sc_params(),
    )
    o1, o2 = k(srcf, dstf, zin, io)
    return (o1.reshape(NC, 2 * HR, CHUNK), o2.reshape(NC, 2 * HR, CHUNK))


def _spmm_body(src_hbm, dst_hbm, tbl_hbm, zin_hbm, out_hbm,
               idx_s8, idx_d8, rows_a, rows_b, acc_sh,
               isem_s, isem_d, gsem_a, gsem_b, ssem_a, ssem_b):
    c = lax.axis_index("c")
    s = lax.axis_index("s")

    @pl.loop(0, RPT // CHUNK)
    def _zero(k):
        pltpu.sync_copy(zin_hbm, acc_sh.at[pl.ds(s * RPT + k * CHUNK, CHUNK)])

    plsc.subcore_barrier()

    row_base = (c * NS + s) * CPW
    bufs = [rows_a, rows_b]
    gsems = [gsem_a, gsem_b]
    ssems = [ssem_a, ssem_b]

    @pl.loop(0, CPW // 8)
    def _blk(b):
        r0 = row_base + b * 8
        i1 = pltpu.async_copy(src_hbm.at[pl.ds(r0, 8)], idx_s8, isem_s)
        i2 = pltpu.async_copy(dst_hbm.at[pl.ds(r0, 8)], idx_d8, isem_d)
        i1.wait()
        i2.wait()
        g = [None, None]
        sct = [None, None]
        for u in range(8):
            bi = u & 1
            if sct[bi] is not None:
                sct[bi].wait()
            g[bi] = pltpu.async_copy(tbl_hbm.at[idx_s8.at[u]], bufs[bi],
                                     gsems[bi])
            if u >= 1:
                pb = 1 - bi
                g[pb].wait()
                sct[pb] = pltpu.async_copy(bufs[pb],
                                           acc_sh.at[idx_d8.at[u - 1]],
                                           ssems[pb], add=True)
        g[1].wait()
        sct[1] = pltpu.async_copy(bufs[1], acc_sh.at[idx_d8.at[7]], ssems[1],
                                  add=True)
        sct[0].wait()
        sct[1].wait()

    plsc.subcore_barrier()

    @pl.loop(0, RPT // CHUNK)
    def _wb(k):
        r0 = s * RPT + k * CHUNK
        pltpu.sync_copy(acc_sh.at[pl.ds(r0, CHUNK)],
                        out_hbm.at[pl.ds(c * NP + r0, CHUNK)])


@jax.jit
def _spmm_call(src2d, dst2d, table, zin):
    k = pl.kernel(
        _spmm_body,
        out_type=jax.ShapeDtypeStruct((NC * NP, CHUNK), jnp.float32),
        mesh=_sc_mesh(),
        scratch_types=[
            pltpu.VMEM((8, CHUNK), jnp.int32),
            pltpu.VMEM((8, CHUNK), jnp.int32),
            pltpu.VMEM((CHUNK, CHUNK), jnp.float32),
            pltpu.VMEM((CHUNK, CHUNK), jnp.float32),
            pltpu.VMEM_SHARED((NP, CHUNK), jnp.float32),
            pltpu.SemaphoreType.DMA,
            pltpu.SemaphoreType.DMA,
            pltpu.SemaphoreType.DMA,
            pltpu.SemaphoreType.DMA,
            pltpu.SemaphoreType.DMA,
            pltpu.SemaphoreType.DMA,
        ],
    )
    return k(src2d, dst2d, table, zin).reshape(NC, NP, CHUNK)


def _deg_block(dp_ref):
    d = dp_ref[0] + dp_ref[1]
    sel = (lax.broadcasted_iota(jnp.int32, (128, 8), 0) // 16
           == lax.broadcasted_iota(jnp.int32, (128, 8), 1)
           ).astype(jnp.float32)
    return jnp.dot(d, sel, preferred_element_type=jnp.float32)


def _scale_rows(x, n38):
    w = x.shape[1]
    return (x.reshape(128, 8, w) * n38[:, :, None]).reshape(1024, w)


def _norm38(dp_ref):
    return lax.rsqrt(jnp.maximum(_deg_block(dp_ref), 1.0))


def _k1_body(dsp_ref, feat_ref, t1_ref):
    ns = _norm38(dsp_ref)
    t1_ref[...] = _scale_rows(feat_ref[...], ns)


@jax.jit
def _k1_call(dsp, featp):
    return pl.pallas_call(
        _k1_body,
        grid=(NP // BLK,),
        in_specs=[
            pl.BlockSpec((NC, 128, 128), lambda i: (0, i, 0)),
            pl.BlockSpec((BLK, 128), lambda i: (i, 0)),
        ],
        out_specs=pl.BlockSpec((BLK, 128), lambda i: (i, 0)),
        out_shape=jax.ShapeDtypeStruct((NP, 128), jnp.float32),
    )(dsp, featp)


def _k2_body(p_ref, dsp_ref, ddp_ref, w1_ref, b1_ref, w2_ref, t2_ref):
    nd = _norm38(ddp_ref)
    ns = _norm38(dsp_ref)
    aggf = _scale_rows(p_ref[0] + p_ref[1], nd)
    h1 = jnp.maximum(
        jnp.dot(aggf, w1_ref[...], preferred_element_type=jnp.float32)
        + b1_ref[...], 0.0)
    x2 = jnp.dot(_scale_rows(h1, ns), w2_ref[...],
                 preferred_element_type=jnp.float32)
    t2_ref[...] = jnp.concatenate([x2, h1], axis=1)


@jax.jit
def _k2_call(p1, dsp, ddp, W1, b1, W2):
    return pl.pallas_call(
        _k2_body,
        grid=(NP // BLK,),
        in_specs=[
            pl.BlockSpec((NC, BLK, 128), lambda i: (0, i, 0)),
            pl.BlockSpec((NC, 128, 128), lambda i: (0, i, 0)),
            pl.BlockSpec((NC, 128, 128), lambda i: (0, i, 0)),
            pl.BlockSpec((128, 64), lambda i: (0, 0)),
            pl.BlockSpec((1, 64), lambda i: (0, 0)),
            pl.BlockSpec((64, 64), lambda i: (0, 0)),
        ],
        out_specs=pl.BlockSpec((BLK, 128), lambda i: (i, 0)),
        out_shape=jax.ShapeDtypeStruct((NP, 128), jnp.float32),
    )(p1, dsp, ddp, W1, b1, W2)


def _k3_body(p_ref, dsp_ref, ddp_ref, b2_ref, w3_ref, t3_ref):
    nd = _norm38(ddp_ref)
    ns = _norm38(dsp_ref)
    a = p_ref[0] + p_ref[1]
    h2 = jnp.maximum(_scale_rows(a[:, 0:64], nd) + b2_ref[...], 0.0)
    x3 = jnp.dot(_scale_rows(h2, ns), w3_ref[...],
                 preferred_element_type=jnp.float32)
    t3_ref[...] = jnp.concatenate([x3, h2], axis=1)


@jax.jit
def _k3_call(p2, dsp, ddp, b2, W3):
    return pl.pallas_call(
        _k3_body,
        grid=(NP // BLK,),
        in_specs=[
            pl.BlockSpec((NC, BLK, 128), lambda i: (0, i, 0)),
            pl.BlockSpec((NC, 128, 128), lambda i: (0, i, 0)),
            pl.BlockSpec((NC, 128, 128), lambda i: (0, i, 0)),
            pl.BlockSpec((1, 64), lambda i: (0, 0)),
            pl.BlockSpec((64, 64), lambda i: (0, 0)),
        ],
        out_specs=pl.BlockSpec((BLK, 128), lambda i: (i, 0)),
        out_shape=jax.ShapeDtypeStruct((NP, 128), jnp.float32),
    )(p2, dsp, ddp, b2, W3)


def _k4_body(p_ref, ddp_ref, b3_ref, wm_ref, t4_ref):
    nd = _norm38(ddp_ref)
    a = p_ref[0] + p_ref[1]
    h3 = jnp.maximum(_scale_rows(a[:, 0:64], nd) + b3_ref[...], 0.0)
    t4_ref[...] = jnp.dot(h3, wm_ref[128:192, :],
                          preferred_element_type=jnp.float32)


@jax.jit
def _k4_call(p3, ddp, b3, Wm):
    return pl.pallas_call(
        _k4_body,
        grid=(NP // BLK,),
        in_specs=[
            pl.BlockSpec((NC, BLK, 128), lambda i: (0, i, 0)),
            pl.BlockSpec((NC, 128, 128), lambda i: (0, i, 0)),
            pl.BlockSpec((1, 64), lambda i: (0, 0)),
            pl.BlockSpec((192, 128), lambda i: (0, 0)),
        ],
        out_specs=pl.BlockSpec((BLK, 128), lambda i: (i, 0)),
        out_shape=jax.ShapeDtypeStruct((NP, 128), jnp.float32),
    )(p3, ddp, b3, Wm)


def _k5_body(p2_ref, p3_ref, p4_ref, wm_ref, bm_ref, out_ref):
    j1 = p2_ref[0, :, 64:128] + p2_ref[1, :, 64:128]
    j2 = p3_ref[0, :, 64:128] + p3_ref[1, :, 64:128]
    sz3 = p4_ref[0] + p4_ref[1]
    out_ref[...] = (
        jnp.dot(j1, wm_ref[0:64, :], preferred_element_type=jnp.float32)
        + jnp.dot(j2, wm_ref[64:128, :], preferred_element_type=jnp.float32)
        + sz3 + bm_ref[...])


@jax.jit
def _k5_call(p2, p3, p4, Wm, bm):
    return pl.pallas_call(
        _k5_body,
        grid=(NP // BLK,),
        in_specs=[
            pl.BlockSpec((NC, BLK, 128), lambda i: (0, i, 0)),
            pl.BlockSpec((NC, BLK, 128), lambda i: (0, i, 0)),
            pl.BlockSpec((NC, BLK, 128), lambda i: (0, i, 0)),
            pl.BlockSpec((192, 128), lambda i: (0, 0)),
            pl.BlockSpec((1, 128), lambda i: (0, 0)),
        ],
        out_specs=pl.BlockSpec((BLK, 128), lambda i: (i, 0)),
        out_shape=jax.ShapeDtypeStruct((NP, 128), jnp.float32),
    )(p2, p3, p4, Wm, bm)


@jax.jit
def kernel(feat, edge_index, W1, b1, W2, b2, W3, b3, Wm, bm):
    src = edge_index[0]
    dst = edge_index[1]
    pad_n = EP - EE
    pad_idx = NN + (jnp.arange(pad_n, dtype=jnp.int32) % (NP - NN))
    src2d = jnp.concatenate([src, pad_idx]).reshape(EP // CHUNK, CHUNK)
    dst2d = jnp.concatenate([dst, pad_idx]).reshape(EP // CHUNK, CHUNK)
    featp = jnp.pad(feat, ((0, NP - NN), (0, 0)))
    zin = jnp.zeros((CHUNK, CHUNK), jnp.float32)
    io = jnp.arange(2 * HR, dtype=jnp.int32)

    dsp, ddp = _deg_call(src2d, dst2d, zin, io)
    t1 = _k1_call(dsp, featp)
    p1 = _spmm_call(src2d, dst2d, t1, zin)
    t2 = _k2_call(p1, dsp, ddp, W1, b1.reshape(1, 64), W2)
    p2 = _spmm_call(src2d, dst2d, t2, zin)
    t3 = _k3_call(p2, dsp, ddp, b2.reshape(1, 64), W3)
    p3 = _spmm_call(src2d, dst2d, t3, zin)
    t4 = _k4_call(p3, ddp, b3.reshape(1, 64), Wm)
    p4 = _spmm_call(src2d, dst2d, t4, zin)
    out = _k5_call(p2, p3, p4, Wm, bm.reshape(1, 128))
    return out[:NN]


# speedup vs baseline: 9.4177x; 1.1058x over previous
import dataclasses

import jax
import jax.numpy as jnp
from jax import lax
from jax.experimental import pallas as pl
from jax.experimental.pallas import tpu as pltpu
from jax.experimental.pallas import tpu_sc as plsc

NN = 10000
NP = 10240
EE = 320000
NC = 2
NS = 16
CHUNK = 128
CPW = 80
EP = NC * NS * CPW * CHUNK
RPT = NP // NS
HR = NP // 16
BLK = 1024


def _sc_mesh():
    return plsc.VectorSubcoreMesh(core_axis_name="c", subcore_axis_name="s")


def _sc_params():
    cp = pltpu.CompilerParams()
    if "needs_layout_passes" in pltpu.CompilerParams.__dataclass_fields__:
        cp = dataclasses.replace(cp, needs_layout_passes=False)
    return cp


def _deg_body(src_hbm, dst_hbm, zin_hbm, io_hbm, outs_hbm, outd_hbm,
              idx_all_s, idx_all_d, iid, iid2, hist, accs_sh, accd_sh,
              zsem, fsem_a, fsem_b):
    c = lax.axis_index("c")
    s = lax.axis_index("s")
    lane = jnp.arange(16, dtype=jnp.int32)
    ones16 = jnp.ones((16,), jnp.float32)

    pltpu.sync_copy(zin_hbm.at[pl.ds(0, 80)], accs_sh.at[pl.ds(s * 80, 80)])
    pltpu.sync_copy(zin_hbm.at[pl.ds(0, 80)], accd_sh.at[pl.ds(s * 80, 80)])
    row_base = (c * NS + s) * CPW
    pltpu.sync_copy(src_hbm.at[pl.ds(row_base, CPW)], idx_all_s)
    pltpu.sync_copy(dst_hbm.at[pl.ds(row_base, CPW)], idx_all_d)
    plsc.subcore_barrier()

    iids = [iid, iid2]
    fsems = [fsem_a, fsem_b]

    def one_hist(idx_all, acc_sel):
        for r in range(2):
            hz = []
            for k in range(HR // CHUNK):
                hz.append(pltpu.async_copy(
                    zin_hbm, hist.at[pl.ds(k * CHUNK, CHUNK)], zsem))
            for h in hz:
                h.wait()

            @pl.loop(0, CPW // 8)
            def _blk(b):
                for jj in range(8):
                    for q in range(8):
                        vec = idx_all[b * 8 + jj, pl.ds(q * 16, 16)]
                        m = vec - (r * 5120)
                        mask = (m >= 0) & (m < 5120)
                        mm = jnp.where(mask, m, 0)
                        vrow = lax.shift_right_logical(mm, 3)
                        vcol = lax.shift_left(lax.bitwise_and(mm, 7), 4) + lane
                        plsc.addupdate_scatter(hist, [vrow, vcol], ones16,
                                               mask=mask)

            fl = [None, None]
            for k in range(HR // CHUNK):
                pk = k & 1
                if fl[pk] is not None:
                    fl[pk].wait()
                pltpu.sync_copy(io_hbm.at[pl.ds(r * HR + k * CHUNK, CHUNK)],
                                iids[pk])
                fl[pk] = pltpu.async_copy(hist.at[pl.ds(k * CHUNK, CHUNK)],
                                          acc_sel.at[iids[pk]], fsems[pk],
                                          add=True)
            fl[0].wait()
            fl[1].wait()

    one_hist(idx_all_s, accs_sh)
    one_hist(idx_all_d, accd_sh)
    plsc.subcore_barrier()

    pltpu.sync_copy(accs_sh.at[pl.ds(s * 80, 80)],
                    outs_hbm.at[pl.ds(c * (2 * HR) + s * 80, 80)])
    pltpu.sync_copy(accd_sh.at[pl.ds(s * 80, 80)],
                    outd_hbm.at[pl.ds(c * (2 * HR) + s * 80, 80)])


@jax.jit
def _deg_call(srcf, dstf, zin, io):
    k = pl.kernel(
        _deg_body,
        out_type=(
            jax.ShapeDtypeStruct((NC * 2 * HR, CHUNK), jnp.float32),
            jax.ShapeDtypeStruct((NC * 2 * HR, CHUNK), jnp.float32),
        ),
        mesh=_sc_mesh(),
        scratch_types=[
            pltpu.VMEM((CPW, CHUNK), jnp.int32),
            pltpu.VMEM((CPW, CHUNK), jnp.int32),
            pltpu.VMEM((CHUNK,), jnp.int32),
            pltpu.VMEM((CHUNK,), jnp.int32),
            pltpu.VMEM((HR, CHUNK), jnp.float32),
            pltpu.VMEM_SHARED((2 * HR, CHUNK), jnp.float32),
            pltpu.VMEM_SHARED((2 * HR, CHUNK), jnp.float32),
            pltpu.SemaphoreType.DMA,
            pltpu.SemaphoreType.DMA,
            pltpu.SemaphoreType.DMA,
        ],
        compiler_params=_sc_params(),
    )
    o1, o2 = k(srcf, dstf, zin, io)
    return (o1.reshape(NC, 2 * HR, CHUNK), o2.reshape(NC, 2 * HR, CHUNK))


def _spmm_body(src_hbm, dst_hbm, tbl_hbm, zin_hbm, out_hbm,
               idx_s8, idx_d8, idx_s8b, idx_d8b, rows_a, rows_b, acc_sh,
               isem_s, isem_d, isem_sb, isem_db,
               gsem_a, gsem_b, ssem_a, ssem_b):
    c = lax.axis_index("c")
    s = lax.axis_index("s")

    @pl.loop(0, RPT // CHUNK)
    def _zero(k):
        pltpu.sync_copy(zin_hbm, acc_sh.at[pl.ds(s * RPT + k * CHUNK, CHUNK)])

    plsc.subcore_barrier()

    row_base = (c * NS + s) * CPW
    bufs = [rows_a, rows_b]
    gsems = [gsem_a, gsem_b]
    ssems = [ssem_a, ssem_b]
    ibufs_s = [idx_s8, idx_s8b]
    ibufs_d = [idx_d8, idx_d8b]
    isems_s = [isem_s, isem_sb]
    isems_d = [isem_d, isem_db]

    nblk = CPW // 8
    ih = [None, None]

    def load_idx(b):
        p = b & 1
        r0 = row_base + b * 8
        h1 = pltpu.async_copy(src_hbm.at[pl.ds(r0, 8)], ibufs_s[p], isems_s[p])
        h2 = pltpu.async_copy(dst_hbm.at[pl.ds(r0, 8)], ibufs_d[p], isems_d[p])
        return (h1, h2)

    ih[0] = load_idx(0)
    g = [None, None]
    sct = [None, None]
    for u in range(CPW):
        b, jj = divmod(u, 8)
        p = b & 1
        if jj == 0:
            ih[p][0].wait()
            ih[p][1].wait()
        bi = u & 1
        if sct[bi] is not None:
            sct[bi].wait()
        if jj == 1 and b + 1 < nblk:
            ih[1 - p] = load_idx(b + 1)
        g[bi] = pltpu.async_copy(tbl_hbm.at[ibufs_s[p].at[jj]], bufs[bi],
                                 gsems[bi])
        if u >= 1:
            pb = 1 - bi
            pbb, pjj = divmod(u - 1, 8)
            g[pb].wait()
            sct[pb] = pltpu.async_copy(bufs[pb],
                                       acc_sh.at[ibufs_d[pbb & 1].at[pjj]],
                                       ssems[pb], add=True)
    g[(CPW - 1) & 1].wait()
    sct[(CPW - 1) & 1] = pltpu.async_copy(
        bufs[(CPW - 1) & 1], acc_sh.at[ibufs_d[(nblk - 1) & 1].at[7]],
        ssems[(CPW - 1) & 1], add=True)
    sct[0].wait()
    sct[1].wait()

    plsc.subcore_barrier()

    @pl.loop(0, RPT // CHUNK)
    def _wb(k):
        r0 = s * RPT + k * CHUNK
        pltpu.sync_copy(acc_sh.at[pl.ds(r0, CHUNK)],
                        out_hbm.at[pl.ds(c * NP + r0, CHUNK)])


@jax.jit
def _spmm_call(src2d, dst2d, table, zin):
    k = pl.kernel(
        _spmm_body,
        out_type=jax.ShapeDtypeStruct((NC * NP, CHUNK), jnp.float32),
        mesh=_sc_mesh(),
        scratch_types=[
            pltpu.VMEM((8, CHUNK), jnp.int32),
            pltpu.VMEM((8, CHUNK), jnp.int32),
            pltpu.VMEM((8, CHUNK), jnp.int32),
            pltpu.VMEM((8, CHUNK), jnp.int32),
            pltpu.VMEM((CHUNK, CHUNK), jnp.float32),
            pltpu.VMEM((CHUNK, CHUNK), jnp.float32),
            pltpu.VMEM_SHARED((NP, CHUNK), jnp.float32),
            pltpu.SemaphoreType.DMA,
            pltpu.SemaphoreType.DMA,
            pltpu.SemaphoreType.DMA,
            pltpu.SemaphoreType.DMA,
            pltpu.SemaphoreType.DMA,
            pltpu.SemaphoreType.DMA,
            pltpu.SemaphoreType.DMA,
            pltpu.SemaphoreType.DMA,
        ],
    )
    return k(src2d, dst2d, table, zin).reshape(NC, NP, CHUNK)


def _deg_block(dp_ref):
    d = dp_ref[0] + dp_ref[1]
    sel = (lax.broadcasted_iota(jnp.int32, (128, 8), 0) // 16
           == lax.broadcasted_iota(jnp.int32, (128, 8), 1)
           ).astype(jnp.float32)
    return jnp.dot(d, sel, preferred_element_type=jnp.float32)


def _scale_rows(x, n38):
    w = x.shape[1]
    return (x.reshape(128, 8, w) * n38[:, :, None]).reshape(1024, w)


def _norm38(dp_ref):
    return lax.rsqrt(jnp.maximum(_deg_block(dp_ref), 1.0))


def _k1_body(dsp_ref, feat_ref, t1_ref):
    ns = _norm38(dsp_ref)
    t1_ref[...] = _scale_rows(feat_ref[...], ns)


@jax.jit
def _k1_call(dsp, featp):
    return pl.pallas_call(
        _k1_body,
        grid=(NP // BLK,),
        in_specs=[
            pl.BlockSpec((NC, 128, 128), lambda i: (0, i, 0)),
            pl.BlockSpec((BLK, 128), lambda i: (i, 0)),
        ],
        out_specs=pl.BlockSpec((BLK, 128), lambda i: (i, 0)),
        out_shape=jax.ShapeDtypeStruct((NP, 128), jnp.float32),
    )(dsp, featp)


def _k2_body(p_ref, dsp_ref, ddp_ref, w1_ref, b1_ref, w2_ref, t2_ref):
    nd = _norm38(ddp_ref)
    ns = _norm38(dsp_ref)
    aggf = _scale_rows(p_ref[0] + p_ref[1], nd)
    h1 = jnp.maximum(
        jnp.dot(aggf, w1_ref[...], preferred_element_type=jnp.float32)
        + b1_ref[...], 0.0)
    x2 = jnp.dot(_scale_rows(h1, ns), w2_ref[...],
                 preferred_element_type=jnp.float32)
    t2_ref[...] = jnp.concatenate([x2, h1], axis=1)


@jax.jit
def _k2_call(p1, dsp, ddp, W1, b1, W2):
    return pl.pallas_call(
        _k2_body,
        grid=(NP // BLK,),
        in_specs=[
            pl.BlockSpec((NC, BLK, 128), lambda i: (0, i, 0)),
            pl.BlockSpec((NC, 128, 128), lambda i: (0, i, 0)),
            pl.BlockSpec((NC, 128, 128), lambda i: (0, i, 0)),
            pl.BlockSpec((128, 64), lambda i: (0, 0)),
            pl.BlockSpec((1, 64), lambda i: (0, 0)),
            pl.BlockSpec((64, 64), lambda i: (0, 0)),
        ],
        out_specs=pl.BlockSpec((BLK, 128), lambda i: (i, 0)),
        out_shape=jax.ShapeDtypeStruct((NP, 128), jnp.float32),
    )(p1, dsp, ddp, W1, b1, W2)


def _k3_body(p_ref, dsp_ref, ddp_ref, b2_ref, w3_ref, t3_ref):
    nd = _norm38(ddp_ref)
    ns = _norm38(dsp_ref)
    a = p_ref[0] + p_ref[1]
    h2 = jnp.maximum(_scale_rows(a[:, 0:64], nd) + b2_ref[...], 0.0)
    x3 = jnp.dot(_scale_rows(h2, ns), w3_ref[...],
                 preferred_element_type=jnp.float32)
    t3_ref[...] = jnp.concatenate([x3, h2], axis=1)


@jax.jit
def _k3_call(p2, dsp, ddp, b2, W3):
    return pl.pallas_call(
        _k3_body,
        grid=(NP // BLK,),
        in_specs=[
            pl.BlockSpec((NC, BLK, 128), lambda i: (0, i, 0)),
            pl.BlockSpec((NC, 128, 128), lambda i: (0, i, 0)),
            pl.BlockSpec((NC, 128, 128), lambda i: (0, i, 0)),
            pl.BlockSpec((1, 64), lambda i: (0, 0)),
            pl.BlockSpec((64, 64), lambda i: (0, 0)),
        ],
        out_specs=pl.BlockSpec((BLK, 128), lambda i: (i, 0)),
        out_shape=jax.ShapeDtypeStruct((NP, 128), jnp.float32),
    )(p2, dsp, ddp, b2, W3)


def _k4_body(p_ref, ddp_ref, b3_ref, wm_ref, t4_ref):
    nd = _norm38(ddp_ref)
    a = p_ref[0] + p_ref[1]
    h3 = jnp.maximum(_scale_rows(a[:, 0:64], nd) + b3_ref[...], 0.0)
    t4_ref[...] = jnp.dot(h3, wm_ref[128:192, :],
                          preferred_element_type=jnp.float32)


@jax.jit
def _k4_call(p3, ddp, b3, Wm):
    return pl.pallas_call(
        _k4_body,
        grid=(NP // BLK,),
        in_specs=[
            pl.BlockSpec((NC, BLK, 128), lambda i: (0, i, 0)),
            pl.BlockSpec((NC, 128, 128), lambda i: (0, i, 0)),
            pl.BlockSpec((1, 64), lambda i: (0, 0)),
            pl.BlockSpec((192, 128), lambda i: (0, 0)),
        ],
        out_specs=pl.BlockSpec((BLK, 128), lambda i: (i, 0)),
        out_shape=jax.ShapeDtypeStruct((NP, 128), jnp.float32),
    )(p3, ddp, b3, Wm)


def _k5_body(p2_ref, p3_ref, p4_ref, wm_ref, bm_ref, out_ref):
    j1 = p2_ref[0, :, 64:128] + p2_ref[1, :, 64:128]
    j2 = p3_ref[0, :, 64:128] + p3_ref[1, :, 64:128]
    sz3 = p4_ref[0] + p4_ref[1]
    out_ref[...] = (
        jnp.dot(j1, wm_ref[0:64, :], preferred_element_type=jnp.float32)
        + jnp.dot(j2, wm_ref[64:128, :], preferred_element_type=jnp.float32)
        + sz3 + bm_ref[...])


@jax.jit
def _k5_call(p2, p3, p4, Wm, bm):
    return pl.pallas_call(
        _k5_body,
        grid=(NP // BLK,),
        in_specs=[
            pl.BlockSpec((NC, BLK, 128), lambda i: (0, i, 0)),
            pl.BlockSpec((NC, BLK, 128), lambda i: (0, i, 0)),
            pl.BlockSpec((NC, BLK, 128), lambda i: (0, i, 0)),
            pl.BlockSpec((192, 128), lambda i: (0, 0)),
            pl.BlockSpec((1, 128), lambda i: (0, 0)),
        ],
        out_specs=pl.BlockSpec((BLK, 128), lambda i: (i, 0)),
        out_shape=jax.ShapeDtypeStruct((NP, 128), jnp.float32),
    )(p2, p3, p4, Wm, bm)


@jax.jit
def kernel(feat, edge_index, W1, b1, W2, b2, W3, b3, Wm, bm):
    src = edge_index[0]
    dst = edge_index[1]
    pad_n = EP - EE
    pad_idx = NN + (jnp.arange(pad_n, dtype=jnp.int32) % (NP - NN))
    src2d = jnp.concatenate([src, pad_idx]).reshape(EP // CHUNK, CHUNK)
    dst2d = jnp.concatenate([dst, pad_idx]).reshape(EP // CHUNK, CHUNK)
    featp = jnp.pad(feat, ((0, NP - NN), (0, 0)))
    zin = jnp.zeros((CHUNK, CHUNK), jnp.float32)
    io = jnp.arange(2 * HR, dtype=jnp.int32)

    dsp, ddp = _deg_call(src2d, dst2d, zin, io)
    t1 = _k1_call(dsp, featp)
    p1 = _spmm_call(src2d, dst2d, t1, zin)
    t2 = _k2_call(p1, dsp, ddp, W1, b1.reshape(1, 64), W2)
    p2 = _spmm_call(src2d, dst2d, t2, zin)
    t3 = _k3_call(p2, dsp, ddp, b2.reshape(1, 64), W3)
    p3 = _spmm_call(src2d, dst2d, t3, zin)
    t4 = _k4_call(p3, ddp, b3.reshape(1, 64), Wm)
    p4 = _spmm_call(src2d, dst2d, t4, zin)
    out = _k5_call(p2, p3, p4, Wm, bm.reshape(1, 128))
    return out[:NN]
